# baseline (speedup 1.0000x reference)
import jax
import jax.numpy as jnp
from jax.experimental import pallas as pl
from jax.experimental.pallas import tpu as pltpu


def _nms_kernel(x_ref, o_ref, s_ref):
    x = x_ref[...]
    zc = jnp.zeros_like(x[:, :, :1])
    m = jnp.maximum(x, jnp.concatenate([x[:, :, 1:], zc], axis=2))
    m = jnp.maximum(m, jnp.concatenate([zc, x[:, :, :-1]], axis=2))
    s_ref[:, 7:8, :] = jnp.zeros_like(m[:, :1, :])
    s_ref[:, 136:137, :] = jnp.zeros_like(m[:, :1, :])
    s_ref[:, 8:136, :] = m
    up = s_ref[:, 7:135, :]
    dn = s_ref[:, 9:137, :]
    lm = jnp.maximum(jnp.maximum(m, up), dn)
    o_ref[...] = jnp.where(x == lm, x, 0.0)


def kernel(points):
    b, c, h, w = points.shape
    flat = points.reshape(b * c, h, w)
    cb = 64
    out = pl.pallas_call(
        _nms_kernel,
        out_shape=jax.ShapeDtypeStruct(flat.shape, flat.dtype),
        grid=(flat.shape[0] // cb,),
        in_specs=[pl.BlockSpec((cb, h, w), lambda i: (i, 0, 0))],
        out_specs=pl.BlockSpec((cb, h, w), lambda i: (i, 0, 0)),
        scratch_shapes=[pltpu.VMEM((cb, 144, w), jnp.float32)],
        compiler_params=pltpu.CompilerParams(
            dimension_semantics=("arbitrary",),
        ),
    )(flat)
    return out.reshape(b, c, h, w)


# speedup vs baseline: 1.0742x; 1.0742x over previous
import jax
import jax.numpy as jnp
from jax import lax
from jax.experimental import pallas as pl
from jax.experimental.pallas import tpu as pltpu


def _nms_kernel(x_ref, o_ref, s_ref):
    cb, h, w = x_ref.shape
    lane = lax.broadcasted_iota(jnp.int32, (1, 1, w), 2)
    s_ref[:, 7:8, :] = jnp.zeros((cb, 1, w), jnp.float32)
    s_ref[:, 8 + h : 9 + h, :] = jnp.zeros((cb, 1, w), jnp.float32)
    ib = 1
    for i0 in range(0, cb, ib):
        x = x_ref[i0 : i0 + ib]
        xl = jnp.where(lane == w - 1, 0.0, pltpu.roll(x, w - 1, axis=2))
        xr = jnp.where(lane == 0, 0.0, pltpu.roll(x, 1, axis=2))
        m = jnp.maximum(jnp.maximum(x, xl), xr)
        s_ref[i0 : i0 + ib, 8 : 8 + h, :] = m
        up = s_ref[i0 : i0 + ib, 7 : 7 + h, :]
        dn = s_ref[i0 : i0 + ib, 9 : 9 + h, :]
        lm = jnp.maximum(jnp.maximum(m, up), dn)
        o_ref[i0 : i0 + ib] = jnp.where(x == lm, x, 0.0)


def kernel(points):
    b, c, h, w = points.shape
    flat = points.reshape(b * c, h, w)
    cb = 128
    out = pl.pallas_call(
        _nms_kernel,
        out_shape=jax.ShapeDtypeStruct(flat.shape, flat.dtype),
        grid=(flat.shape[0] // cb,),
        in_specs=[pl.BlockSpec((cb, h, w), lambda i: (i, 0, 0))],
        out_specs=pl.BlockSpec((cb, h, w), lambda i: (i, 0, 0)),
        scratch_shapes=[pltpu.VMEM((cb, h + 16, w), jnp.float32)],
        compiler_params=pltpu.CompilerParams(
            dimension_semantics=("arbitrary",),
        ),
    )(flat)
    return out.reshape(b, c, h, w)


# speedup vs baseline: 1.1360x; 1.0575x over previous
import jax
import jax.numpy as jnp
from jax import lax
from jax.experimental import pallas as pl
from jax.experimental.pallas import tpu as pltpu


def _nms_kernel(x_ref, o_ref):
    cb, h, w = x_ref.shape
    lane = lax.broadcasted_iota(jnp.int32, (1, 1, w), 2)
    row = lax.broadcasted_iota(jnp.int32, (1, h, 1), 1)
    ib = 1
    for i0 in range(0, cb, ib):
        x = x_ref[i0 : i0 + ib]
        xl = jnp.where(lane == w - 1, 0.0, pltpu.roll(x, w - 1, axis=2))
        xr = jnp.where(lane == 0, 0.0, pltpu.roll(x, 1, axis=2))
        m = jnp.maximum(jnp.maximum(x, xl), xr)
        mu = jnp.where(row == h - 1, 0.0, pltpu.roll(m, h - 1, axis=1))
        md = jnp.where(row == 0, 0.0, pltpu.roll(m, 1, axis=1))
        lm = jnp.maximum(jnp.maximum(m, mu), md)
        o_ref[i0 : i0 + ib] = jnp.where(x == lm, x, 0.0)


def kernel(points):
    b, c, h, w = points.shape
    flat = points.reshape(b * c, h, w)
    cb = 128
    out = pl.pallas_call(
        _nms_kernel,
        out_shape=jax.ShapeDtypeStruct(flat.shape, flat.dtype),
        grid=(flat.shape[0] // cb,),
        in_specs=[pl.BlockSpec((cb, h, w), lambda i: (i, 0, 0))],
        out_specs=pl.BlockSpec((cb, h, w), lambda i: (i, 0, 0)),
        compiler_params=pltpu.CompilerParams(
            dimension_semantics=("arbitrary",),
        ),
    )(flat)
    return out.reshape(b, c, h, w)


# speedup vs baseline: 1.1470x; 1.0097x over previous
import jax
import jax.numpy as jnp
from jax import lax
from jax.experimental import pallas as pl
from jax.experimental.pallas import tpu as pltpu


def _nms_kernel(x_ref, o_ref):
    cb, h, w = x_ref.shape
    lane = lax.broadcasted_iota(jnp.int32, (1, 1, w), 2)
    row = lax.broadcasted_iota(jnp.int32, (1, h, 1), 1)
    ib = 1
    for i0 in range(0, cb, ib):
        x = x_ref[i0 : i0 + ib]
        xl = jnp.where(lane == w - 1, 0.0, pltpu.roll(x, w - 1, axis=2))
        xr = jnp.where(lane == 0, 0.0, pltpu.roll(x, 1, axis=2))
        m = jnp.maximum(jnp.maximum(x, xl), xr)
        mu = jnp.where(row == h - 1, 0.0, pltpu.roll(m, h - 1, axis=1))
        md = jnp.where(row == 0, 0.0, pltpu.roll(m, 1, axis=1))
        lm = jnp.maximum(jnp.maximum(m, mu), md)
        o_ref[i0 : i0 + ib] = jnp.where(x == lm, x, 0.0)


def kernel(points):
    b, c, h, w = points.shape
    flat = points.reshape(b * c, h, w)
    cb = 160
    out = pl.pallas_call(
        _nms_kernel,
        out_shape=jax.ShapeDtypeStruct(flat.shape, flat.dtype),
        grid=(flat.shape[0] // cb,),
        in_specs=[pl.BlockSpec((cb, h, w), lambda i: (i, 0, 0))],
        out_specs=pl.BlockSpec((cb, h, w), lambda i: (i, 0, 0)),
        compiler_params=pltpu.CompilerParams(
            dimension_semantics=("arbitrary",),
        ),
    )(flat)
    return out.reshape(b, c, h, w)


# speedup vs baseline: 1.2644x; 1.1024x over previous
import jax
import jax.numpy as jnp
from jax import lax
from jax.experimental import pallas as pl
from jax.experimental.pallas import tpu as pltpu

_K = 3
_CBC = 64


def _nms_chunk(x, lane, row):
    h, w = x.shape[-2], x.shape[-1]
    xl = jnp.where(lane == w - 1, 0.0, pltpu.roll(x, w - 1, axis=2))
    xr = jnp.where(lane == 0, 0.0, pltpu.roll(x, 1, axis=2))
    m = jnp.maximum(jnp.maximum(x, xl), xr)
    mu = jnp.where(row == h - 1, 0.0, pltpu.roll(m, h - 1, axis=1))
    md = jnp.where(row == 0, 0.0, pltpu.roll(m, 1, axis=1))
    lm = jnp.maximum(jnp.maximum(m, mu), md)
    return jnp.where(x == lm, x, 0.0)


def _nms_kernel(x_hbm, o_hbm, x_buf, o_buf, in_sem, out_sem):
    n, h, w = x_hbm.shape
    n_steps = n // _CBC
    lane = lax.broadcasted_iota(jnp.int32, (1, 1, w), 2)
    row = lax.broadcasted_iota(jnp.int32, (1, h, 1), 1)

    def dma_in(slot, step):
        pltpu.make_async_copy(
            x_hbm.at[pl.ds(step * _CBC, _CBC)], x_buf.at[slot], in_sem.at[slot]
        ).start()

    def wait_in(slot):
        pltpu.make_async_copy(
            x_hbm.at[pl.ds(0, _CBC)], x_buf.at[slot], in_sem.at[slot]
        ).wait()

    def dma_out(slot, step):
        pltpu.make_async_copy(
            o_buf.at[slot], o_hbm.at[pl.ds(step * _CBC, _CBC)], out_sem.at[slot]
        ).start()

    def wait_out(slot):
        pltpu.make_async_copy(
            o_buf.at[slot], o_hbm.at[pl.ds(0, _CBC)], out_sem.at[slot]
        ).wait()

    for s in range(_K - 1):
        dma_in(s, s)

    def body(step, _):
        cur = lax.rem(step, _K)
        @pl.when(step + _K - 1 < n_steps)
        def _():
            dma_in(lax.rem(step + _K - 1, _K), step + _K - 1)
        wait_in(cur)
        @pl.when(step >= _K)
        def _():
            wait_out(cur)
        for i0 in range(0, _CBC, 1):
            x = x_buf[cur, i0 : i0 + 1]
            o_buf[cur, i0 : i0 + 1] = _nms_chunk(x, lane, row)
        dma_out(cur, step)
        return ()

    lax.fori_loop(0, n_steps, body, ())
    for s in range(_K):
        step = n_steps - _K + s
        wait_out(lax.rem(step, _K))


def kernel(points):
    b, c, h, w = points.shape
    flat = points.reshape(b * c, h, w)
    out = pl.pallas_call(
        _nms_kernel,
        out_shape=jax.ShapeDtypeStruct(flat.shape, flat.dtype),
        in_specs=[pl.BlockSpec(memory_space=pl.ANY)],
        out_specs=pl.BlockSpec(memory_space=pl.ANY),
        scratch_shapes=[
            pltpu.VMEM((_K, _CBC, h, w), jnp.float32),
            pltpu.VMEM((_K, _CBC, h, w), jnp.float32),
            pltpu.SemaphoreType.DMA((_K,)),
            pltpu.SemaphoreType.DMA((_K,)),
        ],
    )(flat)
    return out.reshape(b, c, h, w)
